# baseline (speedup 1.0000x reference)
import functools

import jax
import jax.numpy as jnp
from jax import lax
from jax.experimental import pallas as pl
from jax.experimental.pallas import tpu as pltpu
from jax.experimental.pallas import tpu_sc as plsc

VOCAB = 100000
EMB = 64
B = 64
T = 2048

NUM_CORES = 2
NUM_SUBCORES = 16
NW = NUM_CORES * NUM_SUBCORES
FPW = EMB // NW
UNROLL = 8
IG = 4
OG = 2
NQ = B // IG


def _embed_body(tokT_hbm, x_hbm, posT_hbm, out_hbm,
                row_v, idx0_v, idx1_v, pos_v, o0_v, o1_v, pimg_sh,
                sem_r, sem_i, sem_s, sem_f):
    wid = lax.axis_index("s") * NUM_CORES + lax.axis_index("c")
    sid = lax.axis_index("s")
    idx = (idx0_v, idx1_v)
    o = (o0_v, o1_v)

    e0 = wid * FPW
    pltpu.async_copy(tokT_hbm.at[e0, pl.ds(0, VOCAB)], row_v, sem_r)
    pltpu.async_copy(posT_hbm.at[e0, pl.ds(0, T)], pos_v, sem_r)
    pltpu.async_copy(x_hbm.at[pl.ds(0, IG), pl.ds(0, T)], idx0_v, sem_i)

    for f in range(FPW):
        e = wid * FPW + f
        pltpu.make_async_copy(
            tokT_hbm.at[e, pl.ds(0, VOCAB)], row_v, sem_r).wait()
        pltpu.make_async_copy(
            posT_hbm.at[e, pl.ds(0, T)], pos_v, sem_r).wait()
        pltpu.sync_copy(pos_v, pimg_sh.at[sid])

        def group_pair(gp, _):
            for par in range(2):
                q = 2 * gp + par
                b0 = q * IG
                iq = idx[par]

                @pl.when((q < NQ - 1) | (f < FPW - 1))
                def _prefetch():
                    nb = lax.rem(b0 + IG, B)
                    pltpu.async_copy(
                        x_hbm.at[pl.ds(nb, IG), pl.ds(0, T)],
                        idx[1 - par], sem_i)

                pltpu.make_async_copy(
                    x_hbm.at[pl.ds(b0, IG), pl.ds(0, T)], iq, sem_i).wait()

                for half in range(IG // OG):
                    ov = o[half]

                    @pl.when(q >= 1)
                    def _drain():
                        bh = b0 + half * OG
                        pltpu.make_async_copy(
                            ov,
                            out_hbm.at[pl.ds(bh - IG, OG), e, pl.ds(0, T)],
                            sem_s).wait()

                    for sub in range(OG):
                        pltpu.async_copy(pimg_sh.at[sid], ov.at[sub], sem_f)

                for half in range(IG // OG):
                    bh = b0 + half * OG
                    ov = o[half]
                    for sub in range(OG):
                        pltpu.make_async_copy(
                            pimg_sh.at[sid], ov.at[sub], sem_f).wait()

                    for sub in range(OG):
                        s_b = half * OG + sub

                        @plsc.parallel_loop(0, T // 16, unroll=UNROLL)
                        def _gather_loop(i):
                            s = pl.ds(i * 16, 16)
                            gth = plsc.load_gather(row_v, [iq[s_b, s]])
                            plsc.addupdate(ov.at[sub, s], gth)

                    pltpu.async_copy(
                        ov, out_hbm.at[pl.ds(bh, OG), e, pl.ds(0, T)],
                        sem_s)
            return 0

        lax.fori_loop(0, NQ // 2, group_pair, 0)
        if f + 1 < FPW:
            pltpu.async_copy(
                tokT_hbm.at[e + 1, pl.ds(0, VOCAB)], row_v, sem_r)
            pltpu.async_copy(posT_hbm.at[e + 1, pl.ds(0, T)], pos_v, sem_r)
        pltpu.make_async_copy(
            o[0], out_hbm.at[pl.ds(B - IG, OG), e, pl.ds(0, T)],
            sem_s).wait()
        pltpu.make_async_copy(
            o[1], out_hbm.at[pl.ds(B - OG, OG), e, pl.ds(0, T)],
            sem_s).wait()


@jax.jit
def kernel(x, token_embedding_table, position_embedding_table):
    Bv, Tv = x.shape
    tokT = token_embedding_table.T
    posT = position_embedding_table[:Tv].T
    mesh = plsc.VectorSubcoreMesh(core_axis_name="c", subcore_axis_name="s")
    outT = pl.kernel(
        _embed_body,
        mesh=mesh,
        compiler_params=pltpu.CompilerParams(
            use_tc_tiling_on_sc=True, needs_layout_passes=False),
        out_type=jax.ShapeDtypeStruct((Bv, EMB, Tv), jnp.float32),
        scratch_types=[
            pltpu.VMEM((VOCAB,), jnp.float32),
            pltpu.VMEM((IG, T), jnp.int32),
            pltpu.VMEM((IG, T), jnp.int32),
            pltpu.VMEM((T,), jnp.float32),
            pltpu.VMEM((OG, T), jnp.float32),
            pltpu.VMEM((OG, T), jnp.float32),
            pltpu.VMEM_SHARED((NUM_SUBCORES, T), jnp.float32),
            pltpu.SemaphoreType.DMA,
            pltpu.SemaphoreType.DMA,
            pltpu.SemaphoreType.DMA,
            pltpu.SemaphoreType.DMA,
        ],
    )(tokT, x.astype(jnp.int32), posT)
    return outT.transpose(0, 2, 1)


# speedup vs baseline: 1.0633x; 1.0633x over previous
import jax
import jax.numpy as jnp
from jax import lax
from jax.experimental import pallas as pl
from jax.experimental.pallas import tpu as pltpu
from jax.experimental.pallas import tpu_sc as plsc

VOCAB = 100000
EMB = 64
B = 64
T = 2048

NUM_CORES = 2
NUM_SUBCORES = 16
NW = NUM_CORES * NUM_SUBCORES
FPW = EMB // NW
UNROLL = 8
IG = 4
OG = 2
NQ = B // IG


def _embed_body(tokT_hbm, x_hbm, posT_hbm, out_hbm,
                row_v, idx0_v, idx1_v, pos_v, o0_v, o1_v,
                sem_r, sem_i, sem_s):
    wid = lax.axis_index("s") * NUM_CORES + lax.axis_index("c")
    idx = (idx0_v, idx1_v)
    o = (o0_v, o1_v)

    e0 = wid * FPW
    pltpu.async_copy(tokT_hbm.at[e0, pl.ds(0, VOCAB)], row_v, sem_r)
    pltpu.async_copy(posT_hbm.at[e0, pl.ds(0, T)], pos_v, sem_r)
    pltpu.async_copy(x_hbm.at[pl.ds(0, IG), pl.ds(0, T)], idx0_v, sem_i)

    for f in range(FPW):
        e = wid * FPW + f
        pltpu.make_async_copy(
            tokT_hbm.at[e, pl.ds(0, VOCAB)], row_v, sem_r).wait()
        pltpu.make_async_copy(
            posT_hbm.at[e, pl.ds(0, T)], pos_v, sem_r).wait()

        def group_pair(gp, _):
            for par in range(2):
                q = 2 * gp + par
                b0 = q * IG
                iq = idx[par]

                @pl.when((q < NQ - 1) | (f < FPW - 1))
                def _prefetch():
                    nb = lax.rem(b0 + IG, B)
                    pltpu.async_copy(
                        x_hbm.at[pl.ds(nb, IG), pl.ds(0, T)],
                        idx[1 - par], sem_i)

                pltpu.make_async_copy(
                    x_hbm.at[pl.ds(b0, IG), pl.ds(0, T)], iq, sem_i).wait()

                for half in range(IG // OG):
                    bh = b0 + half * OG
                    ov = o[half]

                    @pl.when(q >= 1)
                    def _drain():
                        pltpu.make_async_copy(
                            ov,
                            out_hbm.at[pl.ds(bh - IG, OG), e, pl.ds(0, T)],
                            sem_s).wait()

                    for sub in range(OG):
                        s_b = half * OG + sub

                        @plsc.parallel_loop(0, T // 16, unroll=UNROLL)
                        def _gather_loop(i):
                            s = pl.ds(i * 16, 16)
                            gth = plsc.load_gather(row_v, [iq[s_b, s]])
                            ov[sub, s] = gth + pos_v[s]

                    pltpu.async_copy(
                        ov, out_hbm.at[pl.ds(bh, OG), e, pl.ds(0, T)],
                        sem_s)
            return 0

        lax.fori_loop(0, NQ // 2, group_pair, 0)
        if f + 1 < FPW:
            pltpu.async_copy(
                tokT_hbm.at[e + 1, pl.ds(0, VOCAB)], row_v, sem_r)
            pltpu.async_copy(posT_hbm.at[e + 1, pl.ds(0, T)], pos_v, sem_r)
        pltpu.make_async_copy(
            o[0], out_hbm.at[pl.ds(B - IG, OG), e, pl.ds(0, T)],
            sem_s).wait()
        pltpu.make_async_copy(
            o[1], out_hbm.at[pl.ds(B - OG, OG), e, pl.ds(0, T)],
            sem_s).wait()


@jax.jit
def kernel(x, token_embedding_table, position_embedding_table):
    Bv, Tv = x.shape
    tokT = token_embedding_table.T
    posT = position_embedding_table[:Tv].T
    mesh = plsc.VectorSubcoreMesh(core_axis_name="c", subcore_axis_name="s")
    outT = pl.kernel(
        _embed_body,
        mesh=mesh,
        compiler_params=pltpu.CompilerParams(
            use_tc_tiling_on_sc=True, needs_layout_passes=False),
        out_type=jax.ShapeDtypeStruct((Bv, EMB, Tv), jnp.float32),
        scratch_types=[
            pltpu.VMEM((VOCAB,), jnp.float32),
            pltpu.VMEM((IG, T), jnp.int32),
            pltpu.VMEM((IG, T), jnp.int32),
            pltpu.VMEM((T,), jnp.float32),
            pltpu.VMEM((OG, T), jnp.float32),
            pltpu.VMEM((OG, T), jnp.float32),
            pltpu.SemaphoreType.DMA,
            pltpu.SemaphoreType.DMA,
            pltpu.SemaphoreType.DMA,
        ],
    )(tokT, x.astype(jnp.int32), posT)
    return outT.transpose(0, 2, 1)
